# baseline (speedup 1.0000x reference)
import functools

import jax
import jax.numpy as jnp
from jax import lax
from jax.experimental import pallas as pl
from jax.experimental.pallas import tpu as pltpu
from jax.experimental.pallas import tpu_sc as plsc

_NC = 2
_NS = 16
_NW = _NC * _NS
_LANES = 16


def _make_pool(B, L, V, D):
    bpw = B // _NW
    rpc = 2
    cw = rpc * L
    ch = bpw // rpc
    nsec = 2
    chs = ch // nsec
    nj = D // _LANES

    mesh = plsc.VectorSubcoreMesh(core_axis_name="c", subcore_axis_name="s")

    @functools.partial(
        pl.kernel,
        mesh=mesh,
        out_type=jax.ShapeDtypeStruct((B * D,), jnp.float32),
        scratch_types=[
            pltpu.VMEM((chs, cw), jnp.int32),
            pltpu.VMEM((cw, D), jnp.float32),
            pltpu.VMEM((cw, D), jnp.float32),
            pltpu.VMEM((cw, D), jnp.float32),
            pltpu.VMEM((bpw // 2 * D,), jnp.float32),
            pltpu.SemaphoreType.DMA,
            pltpu.SemaphoreType.DMA,
            pltpu.SemaphoreType.DMA,
        ],
    )
    def pool(ids_hbm, table_hbm, out_hbm,
             idx_v, rows_a, rows_b, rows_c, pooled_v,
             gsem_a, gsem_b, gsem_c):
        cid = lax.axis_index("c")
        sid = lax.axis_index("s")
        wid = sid * _NC + cid
        bufs = (rows_a, rows_b, rows_c)
        gsems = (gsem_a, gsem_b, gsem_c)

        for sec in range(nsec):
            pltpu.sync_copy(ids_hbm.at[wid, sec], idx_v)

            pltpu.make_async_copy(
                table_hbm.at[idx_v.at[0]], rows_a, gsem_a).start()
            pltpu.make_async_copy(
                table_hbm.at[idx_v.at[1]], rows_b, gsem_b).start()

            sec_base = sec * chs
            nit = (chs + 2) // 3

            def step(i, carry):
                g0 = i * 3
                for b in range(3):
                    g = g0 + b
                    fb = (b + 2) % 3

                    @pl.when(g + 2 < chs)
                    def _fire():
                        pltpu.make_async_copy(
                            table_hbm.at[idx_v.at[g + 2]], bufs[fb],
                            gsems[fb]).start()

                    @pl.when(g < chs)
                    def _work():
                        pltpu.make_async_copy(
                            table_hbm.at[idx_v.at[g]], bufs[b],
                            gsems[b]).wait()
                        buf = bufs[b]
                        for r in range(rpc):
                            row = g * rpc + r
                            for jg in range(0, nj, 4):
                                js = range(jg, jg + 4)
                                accs = {
                                    j: buf[r * L, pl.ds(j * _LANES, _LANES)]
                                    for j in js}
                                for t in range(1, L):
                                    for j in js:
                                        accs[j] = accs[j] + buf[
                                            r * L + t,
                                            pl.ds(j * _LANES, _LANES)]
                                for j in js:
                                    pooled_v[pl.ds(row * D + j * _LANES,
                                                   _LANES)] = accs[j]
                return carry

            lax.fori_loop(0, nit, step, 0)

            pltpu.sync_copy(
                pooled_v,
                out_hbm.at[pl.ds((wid * bpw + sec_base * rpc) * D,
                                 bpw // nsec * D)])

    return pool


def _mm_body(x_ref, w_ref, b_ref, o_ref):
    o_ref[...] = jnp.dot(
        x_ref[...], w_ref[...], preferred_element_type=jnp.float32
    ) + b_ref[...]


@jax.jit
def kernel(input_ids, emb_table, fc_w, fc_b):
    B, L = input_ids.shape
    V, D = emb_table.shape
    O = fc_w.shape[1]
    bpw = B // _NW
    rpc = 2
    cw = rpc * L
    ch = bpw // rpc
    nsec = 2
    chs = ch // nsec

    ids = input_ids.astype(jnp.int32).reshape(_NW, nsec, chs, cw)

    pool = _make_pool(B, L, V, D)
    pooled = pool(ids, emb_table).reshape(B, D)

    w_scaled = fc_w * (1.0 / L)

    bm = 1024
    out = pl.pallas_call(
        _mm_body,
        grid=(B // bm,),
        in_specs=[
            pl.BlockSpec((bm, D), lambda i: (i, 0)),
            pl.BlockSpec((D, O), lambda i: (0, 0)),
            pl.BlockSpec((1, O), lambda i: (0, 0)),
        ],
        out_specs=pl.BlockSpec((bm, O), lambda i: (i, 0)),
        out_shape=jax.ShapeDtypeStruct((B, O), jnp.float32),
    )(pooled, w_scaled, fc_b.reshape(1, O))
    return out


# speedup vs baseline: 1.5741x; 1.5741x over previous
import functools

import jax
import jax.numpy as jnp
from jax import lax
from jax.experimental import pallas as pl
from jax.experimental.pallas import tpu as pltpu
from jax.experimental.pallas import tpu_sc as plsc

_NC = 2
_NS = 16
_NW = _NC * _NS


def _make_pool(B, L, V, D):
    bpw = B // _NW
    rpc = 2
    cw = rpc * L
    ch = bpw // rpc
    nsec = 4
    chs = ch // nsec

    mesh = plsc.VectorSubcoreMesh(core_axis_name="c", subcore_axis_name="s")

    @functools.partial(
        pl.kernel,
        mesh=mesh,
        out_type=jax.ShapeDtypeStruct((B, D), jnp.float32),
        scratch_types=[
            pltpu.VMEM((chs, cw), jnp.int32),
            pltpu.VMEM((chs, cw), jnp.int32),
            pltpu.VMEM((cw, D), jnp.float32),
            pltpu.VMEM((cw, D), jnp.float32),
            pltpu.VMEM((cw, D), jnp.float32),
            pltpu.VMEM_SHARED((_NS * bpw, D), jnp.float32),
            pltpu.SemaphoreType.DMA,
            pltpu.SemaphoreType.DMA,
            pltpu.SemaphoreType.DMA,
            pltpu.SemaphoreType.DMA,
            pltpu.SemaphoreType.DMA,
            pltpu.SemaphoreType.DMA,
        ],
    )
    def pool(ids_hbm, table_hbm, dst_hbm, zeros_hbm, out_hbm,
             idx_v, dst_v, rows_a, rows_b, rows_c, pooled_s,
             gsem_a, gsem_b, gsem_c, ssem_a, ssem_b, ssem_c):
        cid = lax.axis_index("c")
        sid = lax.axis_index("s")
        wid = sid * _NC + cid
        bufs = (rows_a, rows_b, rows_c)
        gsems = (gsem_a, gsem_b, gsem_c)
        ssems = (ssem_a, ssem_b, ssem_c)

        pltpu.sync_copy(zeros_hbm, pooled_s.at[pl.ds(sid * bpw, bpw)])

        for sec in range(nsec):
            pltpu.sync_copy(ids_hbm.at[wid, sec], idx_v)
            pltpu.sync_copy(dst_hbm.at[sid, sec], dst_v)

            pltpu.make_async_copy(
                table_hbm.at[idx_v.at[0]], rows_a, gsem_a).start()
            pltpu.make_async_copy(
                table_hbm.at[idx_v.at[1]], rows_b, gsem_b).start()

            nit = (chs + 2) // 3

            def step(i, carry):
                g0 = i * 3
                for b in range(3):
                    g = g0 + b
                    fb = (b + 2) % 3

                    @pl.when((g >= 1) & (g <= chs))
                    def _drain():
                        pltpu.make_async_copy(
                            bufs[fb], pooled_s.at[dst_v.at[g - 1]],
                            ssems[fb]).wait()

                    @pl.when(g + 2 < chs)
                    def _fire():
                        pltpu.make_async_copy(
                            table_hbm.at[idx_v.at[g + 2]], bufs[fb],
                            gsems[fb]).start()

                    @pl.when(g < chs)
                    def _work():
                        pltpu.make_async_copy(
                            table_hbm.at[idx_v.at[g]], bufs[b],
                            gsems[b]).wait()
                        pltpu.async_copy(
                            bufs[b], pooled_s.at[dst_v.at[g]], ssems[b],
                            add=True)
                return carry

            lax.fori_loop(0, nit, step, 0)

        pltpu.sync_copy(pooled_s.at[pl.ds(sid * bpw, bpw)],
                        out_hbm.at[pl.ds(wid * bpw, bpw)])

    return pool


def _mm_body(x_ref, w_ref, b_ref, o_ref):
    o_ref[...] = jnp.dot(
        x_ref[...], w_ref[...], preferred_element_type=jnp.float32
    ) + b_ref[...]


@jax.jit
def kernel(input_ids, emb_table, fc_w, fc_b):
    B, L = input_ids.shape
    V, D = emb_table.shape
    O = fc_w.shape[1]
    bpw = B // _NW
    rpc = 2
    cw = rpc * L
    ch = bpw // rpc
    nsec = 4
    chs = ch // nsec

    ids = input_ids.astype(jnp.int32).reshape(_NW, nsec, chs, cw)
    toks = bpw * L
    local = jnp.arange(toks, dtype=jnp.int32) // L
    dst = (jnp.arange(_NS, dtype=jnp.int32)[:, None] * bpw
           + local[None, :]).reshape(_NS, nsec, chs, cw)
    zeros = jnp.zeros((bpw, D), jnp.float32)

    pool = _make_pool(B, L, V, D)
    pooled = pool(ids, emb_table, dst, zeros)

    w_scaled = fc_w * (1.0 / L)

    bm = 1024
    out = pl.pallas_call(
        _mm_body,
        grid=(B // bm,),
        in_specs=[
            pl.BlockSpec((bm, D), lambda i: (i, 0)),
            pl.BlockSpec((D, O), lambda i: (0, 0)),
            pl.BlockSpec((1, O), lambda i: (0, 0)),
        ],
        out_specs=pl.BlockSpec((bm, O), lambda i: (i, 0)),
        out_shape=jax.ShapeDtypeStruct((B, O), jnp.float32),
    )(pooled, w_scaled, fc_b.reshape(1, O))
    return out


# speedup vs baseline: 2.3380x; 1.4853x over previous
import functools

import jax
import jax.numpy as jnp
from jax import lax
from jax.experimental import pallas as pl
from jax.experimental.pallas import tpu as pltpu
from jax.experimental.pallas import tpu_sc as plsc

_NC = 2
_NS = 16
_NW = _NC * _NS


def _make_pool(B, L, V, D):
    bpw = B // _NW
    blk = 128
    nblk = bpw // blk

    mesh = plsc.VectorSubcoreMesh(core_axis_name="c", subcore_axis_name="s")

    @functools.partial(
        pl.kernel,
        mesh=mesh,
        out_type=jax.ShapeDtypeStruct((B, D), jnp.float32),
        scratch_types=[
            pltpu.VMEM((L, nblk, blk), jnp.int32),
            pltpu.VMEM((blk, D), jnp.float32),
            pltpu.VMEM((blk, D), jnp.float32),
            pltpu.VMEM((blk, D), jnp.float32),
            pltpu.VMEM((blk, D), jnp.float32),
            pltpu.SemaphoreType.DMA,
            pltpu.SemaphoreType.DMA,
            pltpu.SemaphoreType.DMA,
            pltpu.SemaphoreType.DMA,
        ],
    )
    def pool(ids_hbm, table_hbm, out_hbm,
             idx_v, acc0, acc1, acc2, acc3,
             sem0, sem1, sem2, sem3):
        cid = lax.axis_index("c")
        sid = lax.axis_index("s")
        wid = sid * _NC + cid
        accs = (acc0, acc1, acc2, acc3)
        sems = (sem0, sem1, sem2, sem3)

        pltpu.sync_copy(ids_hbm.at[wid], idx_v)

        for k in range(nblk):
            pltpu.make_async_copy(
                table_hbm.at[idx_v.at[0, k]], accs[k], sems[k]).start()

        def step(t, carry):
            for k in range(nblk):
                pltpu.make_async_copy(
                    table_hbm.at[idx_v.at[t - 1, k]], accs[k],
                    sems[k]).wait()
                pltpu.async_copy(
                    table_hbm.at[idx_v.at[t, k]], accs[k], sems[k],
                    add=True)
            return carry

        lax.fori_loop(1, L, step, 0)

        for k in range(nblk):
            pltpu.make_async_copy(
                table_hbm.at[idx_v.at[L - 1, k]], accs[k], sems[k]).wait()
            pltpu.sync_copy(
                accs[k], out_hbm.at[pl.ds(wid * bpw + k * blk, blk)])

    return pool


def _mm_body(x_ref, w_ref, b_ref, o_ref):
    o_ref[...] = jnp.dot(
        x_ref[...], w_ref[...], preferred_element_type=jnp.float32
    ) + b_ref[...]


@jax.jit
def kernel(input_ids, emb_table, fc_w, fc_b):
    B, L = input_ids.shape
    V, D = emb_table.shape
    O = fc_w.shape[1]
    bpw = B // _NW
    blk = 128
    nblk = bpw // blk

    ids = (input_ids.astype(jnp.int32)
           .reshape(_NW, nblk, blk, L)
           .transpose(0, 3, 1, 2))

    pool = _make_pool(B, L, V, D)
    pooled = pool(ids, emb_table)

    w_scaled = fc_w * (1.0 / L)

    bm = 1024
    out = pl.pallas_call(
        _mm_body,
        grid=(B // bm,),
        in_specs=[
            pl.BlockSpec((bm, D), lambda i: (i, 0)),
            pl.BlockSpec((D, O), lambda i: (0, 0)),
            pl.BlockSpec((1, O), lambda i: (0, 0)),
        ],
        out_specs=pl.BlockSpec((bm, O), lambda i: (i, 0)),
        out_shape=jax.ShapeDtypeStruct((B, O), jnp.float32),
    )(pooled, w_scaled, fc_b.reshape(1, O))
    return out


# speedup vs baseline: 2.4006x; 1.0268x over previous
import functools

import jax
import jax.numpy as jnp
from jax import lax
from jax.experimental import pallas as pl
from jax.experimental.pallas import tpu as pltpu
from jax.experimental.pallas import tpu_sc as plsc

_NC = 2
_NS = 16
_NW = _NC * _NS


def _make_pool(B, L, V, D):
    bpw = B // _NW
    blk = 64
    nblk = bpw // blk

    mesh = plsc.VectorSubcoreMesh(core_axis_name="c", subcore_axis_name="s")

    @functools.partial(
        pl.kernel,
        mesh=mesh,
        out_type=jax.ShapeDtypeStruct((B, D), jnp.float32),
        scratch_types=[
            pltpu.VMEM((L, nblk, blk), jnp.int32),
        ] + [pltpu.VMEM((blk, D), jnp.float32)] * 8
          + [pltpu.SemaphoreType.DMA] * 8,
    )
    def pool(ids_hbm, table_hbm, out_hbm, idx_v, *accsems):
        cid = lax.axis_index("c")
        sid = lax.axis_index("s")
        wid = sid * _NC + cid
        accs = accsems[:8]
        sems = accsems[8:]

        pltpu.sync_copy(ids_hbm.at[wid], idx_v)

        for k in range(nblk):
            pltpu.make_async_copy(
                table_hbm.at[idx_v.at[0, k]], accs[k], sems[k]).start()

        def step(t, carry):
            for k in range(nblk):
                pltpu.make_async_copy(
                    table_hbm.at[idx_v.at[t - 1, k]], accs[k],
                    sems[k]).wait()
                pltpu.async_copy(
                    table_hbm.at[idx_v.at[t, k]], accs[k], sems[k],
                    add=True)
            return carry

        lax.fori_loop(1, L, step, 0)

        for k in range(nblk):
            pltpu.make_async_copy(
                table_hbm.at[idx_v.at[L - 1, k]], accs[k], sems[k]).wait()
            pltpu.sync_copy(
                accs[k], out_hbm.at[pl.ds(wid * bpw + k * blk, blk)])

    return pool


def _mm_body(x_ref, w_ref, b_ref, o_ref):
    o_ref[...] = jnp.dot(
        x_ref[...], w_ref[...], preferred_element_type=jnp.float32
    ) + b_ref[...]


@jax.jit
def kernel(input_ids, emb_table, fc_w, fc_b):
    B, L = input_ids.shape
    V, D = emb_table.shape
    O = fc_w.shape[1]
    bpw = B // _NW
    blk = 64
    nblk = bpw // blk

    ids = (input_ids.astype(jnp.int32)
           .reshape(_NW, nblk, blk, L)
           .transpose(0, 3, 1, 2))

    pool = _make_pool(B, L, V, D)
    pooled = pool(ids, emb_table)

    w_scaled = fc_w * (1.0 / L)

    bm = 1024
    out = pl.pallas_call(
        _mm_body,
        grid=(B // bm,),
        in_specs=[
            pl.BlockSpec((bm, D), lambda i: (i, 0)),
            pl.BlockSpec((D, O), lambda i: (0, 0)),
            pl.BlockSpec((1, O), lambda i: (0, 0)),
        ],
        out_specs=pl.BlockSpec((bm, O), lambda i: (i, 0)),
        out_shape=jax.ShapeDtypeStruct((B, O), jnp.float32),
    )(pooled, w_scaled, fc_b.reshape(1, O))
    return out


# speedup vs baseline: 2.4488x; 1.0201x over previous
import functools

import jax
import jax.numpy as jnp
from jax import lax
from jax.experimental import pallas as pl
from jax.experimental.pallas import tpu as pltpu
from jax.experimental.pallas import tpu_sc as plsc

_NC = 2
_NS = 16
_NW = _NC * _NS


def _make_pool(B, L, V, D):
    bpw = B // _NW
    blk = 64
    nblk = bpw // blk

    mesh = plsc.VectorSubcoreMesh(core_axis_name="c", subcore_axis_name="s")

    @functools.partial(
        pl.kernel,
        mesh=mesh,
        out_type=jax.ShapeDtypeStruct((B, D), jnp.float32),
        scratch_types=[
            pltpu.VMEM((L, nblk, blk), jnp.int32),
        ] + [pltpu.VMEM((blk, D), jnp.float32)] * 8
          + [pltpu.SemaphoreType.DMA] * 8,
    )
    def pool(ids_hbm, table_hbm, out_hbm, idx_v, *accsems):
        cid = lax.axis_index("c")
        sid = lax.axis_index("s")
        wid = sid * _NC + cid
        accs = accsems[:8]
        sems = accsems[8:]

        pltpu.sync_copy(ids_hbm.at[wid], idx_v)

        for k in range(nblk):
            pltpu.make_async_copy(
                table_hbm.at[idx_v.at[0, k]], accs[k], sems[k]).start()

        def step(t, carry):
            for k in range(nblk):
                pltpu.make_async_copy(
                    table_hbm.at[idx_v.at[t - 1, k]], accs[k],
                    sems[k]).wait()
                pltpu.async_copy(
                    table_hbm.at[idx_v.at[t, k]], accs[k], sems[k],
                    add=True)
            return carry

        lax.fori_loop(1, L, step, 0)

        for k in range(nblk):
            pltpu.make_async_copy(
                table_hbm.at[idx_v.at[L - 1, k]], accs[k], sems[k]).wait()
            pltpu.sync_copy(
                accs[k], out_hbm.at[pl.ds(wid * bpw + k * blk, blk)])

    return pool


def _mm_body(x_ref, w_ref, b_ref, o_ref):
    o_ref[...] = jnp.dot(
        x_ref[...], w_ref[...], preferred_element_type=jnp.float32
    ) + b_ref[...]


@jax.jit
def kernel(input_ids, emb_table, fc_w, fc_b):
    B, L = input_ids.shape
    V, D = emb_table.shape
    O = fc_w.shape[1]
    bpw = B // _NW
    blk = 64
    nblk = bpw // blk

    ids = (input_ids.astype(jnp.int32)
           .reshape(_NW, nblk, blk, L)
           .transpose(0, 3, 1, 2))

    pool = _make_pool(B, L, V, D)
    pooled = pool(ids, emb_table)

    w_scaled = fc_w * (1.0 / L)

    bm = 2048
    out = pl.pallas_call(
        _mm_body,
        grid=(B // bm,),
        in_specs=[
            pl.BlockSpec((bm, D), lambda i: (i, 0)),
            pl.BlockSpec((D, O), lambda i: (0, 0)),
            pl.BlockSpec((1, O), lambda i: (0, 0)),
        ],
        out_specs=pl.BlockSpec((bm, O), lambda i: (i, 0)),
        out_shape=jax.ShapeDtypeStruct((B, O), jnp.float32),
    )(pooled, w_scaled, fc_b.reshape(1, O))
    return out
